# baseline (speedup 1.0000x reference)
import functools

import jax
import jax.numpy as jnp
from jax import lax
from jax.experimental import pallas as pl
from jax.experimental.pallas import tpu as pltpu
from jax.experimental.pallas import tpu_sc as plsc

NC = 2
NS = 16
L = 16
NBUF = 4
NW = 8


def _mm1_body(x_ref, w_ref, o_ref):
    o_ref[...] = jnp.dot(x_ref[...], w_ref[...],
                         preferred_element_type=jnp.float32)


def _mm2_body(a_ref, w_ref, o_ref):
    xv = jnp.maximum(a_ref[...], 0.0)
    o_ref[...] = jnp.dot(xv, w_ref[...], preferred_element_type=jnp.float32)


def _matmul(body, x, w, block_rows=1000):
    n, k = x.shape
    m = w.shape[1]
    return pl.pallas_call(
        body,
        grid=(n // block_rows,),
        in_specs=[
            pl.BlockSpec((block_rows, k), lambda i: (i, 0)),
            pl.BlockSpec((k, m), lambda i: (0, 0)),
        ],
        out_specs=pl.BlockSpec((block_rows, m), lambda i: (i, 0)),
        out_shape=jax.ShapeDtypeStruct((n, m), jnp.float32),
    )(x, w)


def _sc_aggregate(table, comb, binit, *, npc, sl, chunk, ch, nbuf=NBUF, nw=NW):
    mesh = plsc.VectorSubcoreMesh(core_axis_name="c", subcore_axis_name="s")
    npadc = -(-npc // (NS * 8)) * (NS * 8)
    rps = npadc // NS
    ka = nw - nbuf + 1
    assert ch % nw == 0 and ch >= nw and chunk % L == 0
    assert nw % nbuf == 0 and ka >= 2

    @functools.partial(
        pl.kernel,
        mesh=mesh,
        out_type=jax.ShapeDtypeStruct((2, npadc, sl, 128), jnp.float32),
        scratch_types=[
            pltpu.VMEM((nw, 3, chunk), jnp.int32),
            pltpu.VMEM((nbuf, chunk, sl, 128), jnp.float32),
            pltpu.VMEM((sl, 128), jnp.float32),
            pltpu.VMEM_SHARED((npadc, sl, 128), jnp.float32),
        ] + [pltpu.SemaphoreType.DMA] * (2 * nbuf + nw),
    )
    def agg_kernel(tab_h, comb_h, b_h, out_h,
                   comb_v, rows_v, b_v, acc, *sems):
        gsem = sems[:nbuf]
        ssem = sems[nbuf:2 * nbuf]
        csem = sems[2 * nbuf:]
        c = lax.axis_index("c")
        s = lax.axis_index("s")
        pltpu.sync_copy(b_h.at[c], b_v)

        def fill_row(r, _):
            for h in range(sl):
                for kk in range(128 // L):
                    rows_v[0, r, h, pl.ds(kk * L, L)] = b_v[h, pl.ds(kk * L, L)]
            return 0
        lax.fori_loop(0, chunk, fill_row, 0)
        r0 = s * rps
        off = 0
        while off < rps:
            nr = min(chunk, rps - off)
            pltpu.sync_copy(rows_v.at[0, pl.ds(0, nr)],
                            acc.at[pl.ds(r0 + off, nr)])
            off += nr
        plsc.subcore_barrier()

        bcast_dnums = lax.GatherDimensionNumbers(
            offset_dims=(), collapsed_slice_dims=(0,), start_index_map=(0,))

        def comb_load(j, w):
            jc = jnp.minimum(j, ch - 1)
            pltpu.async_copy(comb_h.at[c, s, jc], comb_v.at[w], csem[w])

        def comb_wait(w):
            pltpu.make_async_copy(comb_h.at[0, 0, 0], comb_v.at[w],
                                  csem[w]).wait()

        def gather_start(b, w):
            pltpu.async_copy(tab_h.at[comb_v.at[w, 0]], rows_v.at[b],
                             gsem[b])

        def gather_wait(b):
            pltpu.make_async_copy(tab_h.at[comb_v.at[0, 0]], rows_v.at[b],
                                  gsem[b]).wait()

        def scatter_start(b, w):
            pltpu.async_copy(rows_v.at[b], acc.at[comb_v.at[w, 1]],
                             ssem[b], add=True)

        def scatter_wait(b):
            pltpu.make_async_copy(rows_v.at[b], acc.at[comb_v.at[0, 1]],
                                  ssem[b]).wait()

        def scale_rows(b, w):
            def group_body(g, _):
                eav16 = lax.bitcast_convert_type(
                    comb_v[w, 2, pl.ds(g * L, L)], jnp.float32)
                for t in range(L):
                    eab = lax.gather(
                        eav16, jnp.full((L, 1), t, jnp.int32), bcast_dnums,
                        (1,), mode=lax.GatherScatterMode.PROMISE_IN_BOUNDS)
                    i = g * L + t
                    for h in range(sl):
                        rv = rows_v.at[b, i, h]
                        for kk in range(128 // L):
                            rv[pl.ds(kk * L, L)] = rv[pl.ds(kk * L, L)] * eab
                return 0
            lax.fori_loop(0, chunk // L, group_body, 0)

        def position(q, qq):
            b, w = qq % nbuf, qq % nw
            if not (isinstance(q, int) and q < nbuf - 1):
                scatter_wait((qq + 1) % nbuf)
            comb_load(q + ka, (qq + ka) % nw)
            comb_wait((qq + 1) % nw)
            gather_start((qq + 1) % nbuf, (qq + 1) % nw)
            gather_wait(b)
            scale_rows(b, w)
            scatter_start(b, w)

        for j in range(ka):
            comb_load(j, j)
        comb_wait(0)
        gather_start(0, 0)
        for qq in range(nw):
            position(qq, qq)

        def cycle_body(k, _):
            for qq in range(nw):
                position(k * nw + qq, qq)
            return 0
        lax.fori_loop(1, ch // nw, cycle_body, 0)

        for w in range(1, ka):
            comb_wait(w)
        gather_wait(0)
        for b in range(1, nbuf):
            scatter_wait(b)
        plsc.subcore_barrier()
        pltpu.sync_copy(acc.at[pl.ds(r0, rps)], out_h.at[c, pl.ds(r0, rps)])

    return agg_kernel(table, comb, binit)


def kernel(x, edge_index, edge_attr, W1, b1, W2, b2):
    n = x.shape[0]
    half = n // 2
    src = edge_index[0].astype(jnp.int32)
    dst = edge_index[1].astype(jnp.int32)
    eab = jax.lax.bitcast_convert_type(edge_attr.astype(jnp.float32),
                                       jnp.int32)
    e = src.shape[0]

    CH1, CH2 = 128, 64
    grain = NS * CH1 * NW
    EPC = -(-(e // 2 + 1600) // grain) * grain
    nch1 = EPC // (NS * CH1)
    nch2 = EPC // (NS * CH2)

    key = (dst >= half).astype(jnp.int32)
    pos0 = jnp.cumsum(1 - key) - 1
    pos1 = jnp.cumsum(key) - 1
    pos = jnp.where(key == 0, pos0, EPC + pos1)
    src_p = jnp.zeros((2 * EPC,), jnp.int32).at[pos].set(src, mode="drop")
    dst_p = jnp.zeros((2 * EPC,), jnp.int32).at[pos].set(dst - half * key,
                                                         mode="drop")
    ea_p = jnp.zeros((2 * EPC,), jnp.int32).at[pos].set(eab, mode="drop")

    def mk_comb(nch, chk):
        sh = (NC, NS, nch, chk)
        return jnp.stack([src_p.reshape(sh), dst_p.reshape(sh),
                          ea_p.reshape(sh)], axis=3)

    comb1 = mk_comb(nch1, CH1)
    comb2 = mk_comb(nch2, CH2)
    b1init = jnp.stack([b1, b1])
    b2init = jnp.stack([b2, b2])

    d1, d2 = W1.shape[1], W2.shape[1]
    h1 = _matmul(_mm1_body, x, W1)
    parts = _sc_aggregate(h1.reshape(n, d1 // 128, 128), comb1,
                          b1init.reshape(NC, d1 // 128, 128), npc=half,
                          sl=d1 // 128, chunk=CH1, ch=nch1)
    agg1 = jnp.concatenate([parts[0, :half], parts[1, :half]]).reshape(n, d1)
    h2 = _matmul(_mm2_body, agg1, W2)
    out = _sc_aggregate(h2.reshape(n, d2 // 128, 128), comb2,
                        b2init.reshape(NC, d2 // 128, 128), npc=half,
                        sl=d2 // 128, chunk=CH2, ch=nch2, nbuf=2, nw=4)
    return jnp.concatenate([out[0, :half], out[1, :half]]).reshape(n, d2)


# speedup vs baseline: 2.7165x; 2.7165x over previous
import functools

import jax
import jax.numpy as jnp
from jax import lax
from jax.experimental import pallas as pl
from jax.experimental.pallas import tpu as pltpu
from jax.experimental.pallas import tpu_sc as plsc

NC = 2
NS = 16
L = 16
NBUF = 4
NW = 8


def _mm1_body(x_ref, w_ref, o_ref):
    o_ref[...] = jnp.dot(x_ref[...], w_ref[...],
                         preferred_element_type=jnp.float32)


def _mm2_body(a_ref, w_ref, o_ref):
    xv = jnp.maximum(a_ref[...], 0.0)
    o_ref[...] = jnp.dot(xv, w_ref[...], preferred_element_type=jnp.float32)


def _matmul(body, x, w, block_rows=1000):
    n, k = x.shape
    m = w.shape[1]
    return pl.pallas_call(
        body,
        grid=(n // block_rows,),
        in_specs=[
            pl.BlockSpec((block_rows, k), lambda i: (i, 0)),
            pl.BlockSpec((k, m), lambda i: (0, 0)),
        ],
        out_specs=pl.BlockSpec((block_rows, m), lambda i: (i, 0)),
        out_shape=jax.ShapeDtypeStruct((n, m), jnp.float32),
    )(x, w)


def _sc_aggregate(table, comb, binit, *, npc, sl, chunk, ch, nbuf=NBUF, nw=NW):
    mesh = plsc.VectorSubcoreMesh(core_axis_name="c", subcore_axis_name="s")
    npadc = -(-npc // (NS * 8)) * (NS * 8)
    rps = npadc // NS
    ka = nw - nbuf + 1
    assert ch % nw == 0 and ch >= nw and chunk % L == 0
    assert nw % nbuf == 0 and ka >= 2

    @functools.partial(
        pl.kernel,
        mesh=mesh,
        out_type=jax.ShapeDtypeStruct((2, npadc, sl, 128), jnp.float32),
        scratch_types=[
            pltpu.VMEM((nw, 3, chunk), jnp.int32),
            pltpu.VMEM((nbuf, chunk, sl, 128), jnp.float32),
            pltpu.VMEM((sl, 128), jnp.float32),
            pltpu.VMEM_SHARED((npadc, sl, 128), jnp.float32),
        ] + [pltpu.SemaphoreType.DMA] * (2 * nbuf + nw),
    )
    def agg_kernel(tab_h, comb_h, b_h, out_h,
                   comb_v, rows_v, b_v, acc, *sems):
        gsem = sems[:nbuf]
        ssem = sems[nbuf:2 * nbuf]
        csem = sems[2 * nbuf:]
        c = lax.axis_index("c")
        s = lax.axis_index("s")
        pltpu.sync_copy(b_h.at[c], b_v)

        def fill_row(r, _):
            for h in range(sl):
                for kk in range(128 // L):
                    rows_v[0, r, h, pl.ds(kk * L, L)] = b_v[h, pl.ds(kk * L, L)]
            return 0
        lax.fori_loop(0, chunk, fill_row, 0)
        r0 = s * rps
        off = 0
        while off < rps:
            nr = min(chunk, rps - off)
            pltpu.sync_copy(rows_v.at[0, pl.ds(0, nr)],
                            acc.at[pl.ds(r0 + off, nr)])
            off += nr
        plsc.subcore_barrier()

        bcast_dnums = lax.GatherDimensionNumbers(
            offset_dims=(), collapsed_slice_dims=(0,), start_index_map=(0,))

        def comb_load(j, w):
            jc = jnp.minimum(j, ch - 1)
            pltpu.async_copy(comb_h.at[c, s, jc], comb_v.at[w], csem[w])

        def comb_wait(w):
            pltpu.make_async_copy(comb_h.at[0, 0, 0], comb_v.at[w],
                                  csem[w]).wait()

        def gather_start(b, w):
            pltpu.async_copy(tab_h.at[comb_v.at[w, 0]], rows_v.at[b],
                             gsem[b])

        def gather_wait(b):
            pltpu.make_async_copy(tab_h.at[comb_v.at[0, 0]], rows_v.at[b],
                                  gsem[b]).wait()

        def scatter_start(b, w):
            pltpu.async_copy(rows_v.at[b], acc.at[comb_v.at[w, 1]],
                             ssem[b], add=True)

        def scatter_wait(b):
            pltpu.make_async_copy(rows_v.at[b], acc.at[comb_v.at[0, 1]],
                                  ssem[b]).wait()

        def scale_rows(b, w):
            def group_body(g, _):
                eav16 = lax.bitcast_convert_type(
                    comb_v[w, 2, pl.ds(g * L, L)], jnp.float32)
                for t in range(L):
                    eab = lax.gather(
                        eav16, jnp.full((L, 1), t, jnp.int32), bcast_dnums,
                        (1,), mode=lax.GatherScatterMode.PROMISE_IN_BOUNDS)
                    i = g * L + t
                    for h in range(sl):
                        rv = rows_v.at[b, i, h]
                        for kk in range(128 // L):
                            rv[pl.ds(kk * L, L)] = rv[pl.ds(kk * L, L)] * eab
                return 0
            lax.fori_loop(0, chunk // L, group_body, 0)

        def position(q, qq):
            b, w = qq % nbuf, qq % nw
            if not (isinstance(q, int) and q < nbuf - 1):
                scatter_wait((qq + 1) % nbuf)
            comb_load(q + ka, (qq + ka) % nw)
            comb_wait((qq + 1) % nw)
            gather_start((qq + 1) % nbuf, (qq + 1) % nw)
            gather_wait(b)
            scale_rows(b, w)
            scatter_start(b, w)

        for j in range(ka):
            comb_load(j, j)
        comb_wait(0)
        gather_start(0, 0)
        for qq in range(nw):
            position(qq, qq)

        def cycle_body(k, _):
            for qq in range(nw):
                position(k * nw + qq, qq)
            return 0
        lax.fori_loop(1, ch // nw, cycle_body, 0)

        for w in range(1, ka):
            comb_wait(w)
        gather_wait(0)
        for b in range(1, nbuf):
            scatter_wait(b)
        plsc.subcore_barrier()
        pltpu.sync_copy(acc.at[pl.ds(r0, rps)], out_h.at[c, pl.ds(r0, rps)])

    return agg_kernel(table, comb, binit)


def kernel(x, edge_index, edge_attr, W1, b1, W2, b2):
    n = x.shape[0]
    half = n // 2
    src = edge_index[0].astype(jnp.int32)
    dst = edge_index[1].astype(jnp.int32)
    eab = jax.lax.bitcast_convert_type(edge_attr.astype(jnp.float32),
                                       jnp.int32)
    e = src.shape[0]

    CH1, CH2 = 128, 64
    grain = NS * CH1 * NW
    EPC = -(-(e // 2 + 1600) // grain) * grain
    nch1 = EPC // (NS * CH1)
    nch2 = EPC // (NS * CH2)

    key = (dst >= half).astype(jnp.int32)
    pad_n = 2 * EPC - e
    count0 = jnp.sum(1 - key)
    pad_key = (jnp.arange(pad_n, dtype=jnp.int32) >= EPC - count0)
    key_f = jnp.concatenate([key, pad_key.astype(jnp.int32)])
    zpad = jnp.zeros((pad_n,), jnp.int32)
    _, src_p, dst_p, ea_p = lax.sort(
        (key_f, jnp.concatenate([src, zpad]),
         jnp.concatenate([dst - half * key, zpad]),
         jnp.concatenate([eab, zpad])), num_keys=1, is_stable=True)

    def mk_comb(nch, chk):
        sh = (NC, NS, nch, chk)
        return jnp.stack([src_p.reshape(sh), dst_p.reshape(sh),
                          ea_p.reshape(sh)], axis=3)

    comb1 = mk_comb(nch1, CH1)
    comb2 = mk_comb(nch2, CH2)
    b1init = jnp.stack([b1, b1])
    b2init = jnp.stack([b2, b2])

    d1, d2 = W1.shape[1], W2.shape[1]
    h1 = _matmul(_mm1_body, x, W1)
    parts = _sc_aggregate(h1.reshape(n, d1 // 128, 128), comb1,
                          b1init.reshape(NC, d1 // 128, 128), npc=half,
                          sl=d1 // 128, chunk=CH1, ch=nch1)
    agg1 = jnp.concatenate([parts[0, :half], parts[1, :half]]).reshape(n, d1)
    h2 = _matmul(_mm2_body, agg1, W2)
    out = _sc_aggregate(h2.reshape(n, d2 // 128, 128), comb2,
                        b2init.reshape(NC, d2 // 128, 128), npc=half,
                        sl=d2 // 128, chunk=CH2, ch=nch2, nbuf=2, nw=4)
    return jnp.concatenate([out[0, :half], out[1, :half]]).reshape(n, d2)


# speedup vs baseline: 3.4656x; 1.2758x over previous
import functools

import jax
import jax.numpy as jnp
from jax import lax
from jax.experimental import pallas as pl
from jax.experimental.pallas import tpu as pltpu
from jax.experimental.pallas import tpu_sc as plsc

NC = 2
NS = 16
L = 16
CHUNK = 64
DH = 128
NBUF = 4
NW = 8


def _mm1_body(x_ref, w_ref, o_ref):
    o_ref[...] = jnp.dot(x_ref[...], w_ref[...],
                         preferred_element_type=jnp.float32)


def _mm2_body(p0_ref, p1_ref, w_ref, o_ref):
    xv = jnp.maximum(p0_ref[...] + p1_ref[...], 0.0)
    o_ref[...] = jnp.dot(xv, w_ref[...], preferred_element_type=jnp.float32)


def _matmul1(x, w, block_rows=1000):
    n, k = x.shape
    m = w.shape[1]
    return pl.pallas_call(
        _mm1_body,
        grid=(n // block_rows,),
        in_specs=[
            pl.BlockSpec((block_rows, k), lambda i: (i, 0)),
            pl.BlockSpec((k, m), lambda i: (0, 0)),
        ],
        out_specs=pl.BlockSpec((block_rows, m), lambda i: (i, 0)),
        out_shape=jax.ShapeDtypeStruct((n, m), jnp.float32),
    )(x, w)


def _matmul2(p0, p1, w, block_rows=1000):
    n, k = p0.shape
    m = w.shape[1]
    return pl.pallas_call(
        _mm2_body,
        grid=(n // block_rows,),
        in_specs=[
            pl.BlockSpec((block_rows, k), lambda i: (i, 0)),
            pl.BlockSpec((block_rows, k), lambda i: (i, 0)),
            pl.BlockSpec((k, m), lambda i: (0, 0)),
        ],
        out_specs=pl.BlockSpec((block_rows, m), lambda i: (i, 0)),
        out_shape=jax.ShapeDtypeStruct((n, m), jnp.float32),
    )(p0, p1, w)


def _sc_aggregate(table, comb, binit, *, n_nodes, ch):
    mesh = plsc.VectorSubcoreMesh(core_axis_name="c", subcore_axis_name="s")
    npad = -(-n_nodes // (NS * 8)) * (NS * 8)
    rps = npad // NS
    assert ch % NW == 0 and ch >= NW

    @functools.partial(
        pl.kernel,
        mesh=mesh,
        out_type=jax.ShapeDtypeStruct((2, npad, DH), jnp.float32),
        scratch_types=[
            pltpu.VMEM((NW, 3, CHUNK), jnp.int32),
            pltpu.VMEM((NBUF, CHUNK, DH), jnp.float32),
            pltpu.VMEM((DH,), jnp.float32),
            pltpu.VMEM_SHARED((npad, DH), jnp.float32),
        ] + [pltpu.SemaphoreType.DMA] * (2 * NBUF + NW),
    )
    def agg_kernel(tab_h, comb_h, b_h, out_h,
                   comb_v, rows_v, b_v, acc, *sems):
        gsem = sems[:NBUF]
        ssem = sems[NBUF:2 * NBUF]
        csem = sems[2 * NBUF:]
        c = lax.axis_index("c")
        s = lax.axis_index("s")
        pltpu.sync_copy(b_h.at[c], b_v)

        def fill_row(r, _):
            for kk in range(DH // L):
                rows_v[0, r, pl.ds(kk * L, L)] = b_v[pl.ds(kk * L, L)]
            return 0
        lax.fori_loop(0, CHUNK, fill_row, 0)
        r0 = s * rps
        off = 0
        while off < rps:
            nr = min(CHUNK, rps - off)
            pltpu.sync_copy(rows_v.at[0, pl.ds(0, nr)],
                            acc.at[pl.ds(r0 + off, nr)])
            off += nr
        plsc.subcore_barrier()

        bcast_dnums = lax.GatherDimensionNumbers(
            offset_dims=(), collapsed_slice_dims=(0,), start_index_map=(0,))

        def comb_load(j, w):
            jc = jnp.minimum(j, ch - 1)
            pltpu.async_copy(comb_h.at[c, s, jc], comb_v.at[w], csem[w])

        def comb_wait(w):
            pltpu.make_async_copy(comb_h.at[0, 0, 0], comb_v.at[w],
                                  csem[w]).wait()

        def gather_start(b, w):
            pltpu.async_copy(tab_h.at[comb_v.at[w, 0]], rows_v.at[b],
                             gsem[b])

        def gather_wait(b):
            pltpu.make_async_copy(tab_h.at[comb_v.at[0, 0]], rows_v.at[b],
                                  gsem[b]).wait()

        def scatter_start(b, w):
            pltpu.async_copy(rows_v.at[b], acc.at[comb_v.at[w, 1]],
                             ssem[b], add=True)

        def scatter_wait(b):
            pltpu.make_async_copy(rows_v.at[b], acc.at[comb_v.at[0, 1]],
                                  ssem[b]).wait()

        def scale_rows(b, w):
            def group_body(g, _):
                eav16 = lax.bitcast_convert_type(
                    comb_v[w, 2, pl.ds(g * L, L)], jnp.float32)
                for t in range(L):
                    eab = lax.gather(
                        eav16, jnp.full((L, 1), t, jnp.int32), bcast_dnums,
                        (1,), mode=lax.GatherScatterMode.PROMISE_IN_BOUNDS)
                    i = g * L + t
                    rv = rows_v.at[b, i]
                    for kk in range(DH // L):
                        rv[pl.ds(kk * L, L)] = rv[pl.ds(kk * L, L)] * eab
                return 0
            lax.fori_loop(0, CHUNK // L, group_body, 0)

        def position(q, qq):
            b, w = qq % NBUF, qq % NW
            if not (isinstance(q, int) and q < 3):
                scatter_wait((qq + 1) % NBUF)
            comb_load(q + 5, (qq + 5) % NW)
            comb_wait((qq + 1) % NW)
            gather_start((qq + 1) % NBUF, (qq + 1) % NW)
            gather_wait(b)
            scale_rows(b, w)
            scatter_start(b, w)

        for j in range(5):
            comb_load(j, j)
        comb_wait(0)
        gather_start(0, 0)
        for qq in range(NW):
            position(qq, qq)

        def cycle_body(k, _):
            for qq in range(NW):
                position(k * NW + qq, qq)
            return 0
        lax.fori_loop(1, ch // NW, cycle_body, 0)

        for w in range(1, 5):
            comb_wait(w)
        gather_wait(0)
        for b in range(1, NBUF):
            scatter_wait(b)
        plsc.subcore_barrier()
        pltpu.sync_copy(acc.at[pl.ds(r0, rps)], out_h.at[c, pl.ds(r0, rps)])

    return agg_kernel(table, comb, binit)


def _pad_reshape(a, ep, shape):
    return jnp.pad(a, (0, ep - a.shape[0])).reshape(shape)


def kernel(x, edge_index, edge_attr, W1, b1, W2, b2):
    n = x.shape[0]
    d_hid = W1.shape[1]
    d_out = W2.shape[1]
    src = edge_index[0].astype(jnp.int32)
    dst = edge_index[1].astype(jnp.int32)
    ea = edge_attr.astype(jnp.float32)
    e = src.shape[0]

    eab = jax.lax.bitcast_convert_type(ea, jnp.int32)

    ep1 = -(-e // (NC * NS * CHUNK * NW)) * (NC * NS * CHUNK * NW)
    ch1 = ep1 // (NC * NS * CHUNK)
    sh1 = (NC, NS, ch1, CHUNK)
    comb1 = jnp.stack([_pad_reshape(src, ep1, sh1),
                       _pad_reshape(dst, ep1, sh1),
                       _pad_reshape(eab, ep1, sh1)], axis=3)
    b1init = jnp.stack([b1, jnp.zeros_like(b1)])

    ep2 = -(-e // (NS * CHUNK * NW)) * (NS * CHUNK * NW)
    ch2 = ep2 // (NS * CHUNK)
    sh2 = (NS, ch2, CHUNK)
    src2 = _pad_reshape(src, ep2, sh2)
    idx2 = 2 * src2[None] + jnp.arange(NC, dtype=jnp.int32)[:, None, None, None]
    dst2 = jnp.broadcast_to(_pad_reshape(dst, ep2, sh2)[None], (NC,) + sh2)
    ea2 = jnp.broadcast_to(_pad_reshape(eab, ep2, sh2)[None], (NC,) + sh2)
    comb2 = jnp.stack([idx2, dst2, ea2], axis=3)
    b2init = b2.reshape(NC, d_out // NC)

    h1 = _matmul1(x, W1)
    parts = _sc_aggregate(h1, comb1, b1init, n_nodes=n, ch=ch1)
    h2 = _matmul2(parts[0, :n], parts[1, :n], W2)
    halves = _sc_aggregate(h2.reshape(2 * n, d_out // 2), comb2,
                           b2init, n_nodes=n, ch=ch2)
    return halves[:, :n, :].transpose(1, 0, 2).reshape(n, d_out)


# speedup vs baseline: 3.5483x; 1.0238x over previous
import functools

import jax
import jax.numpy as jnp
from jax import lax
from jax.experimental import pallas as pl
from jax.experimental.pallas import tpu as pltpu
from jax.experimental.pallas import tpu_sc as plsc

NC = 2
NS = 16
L = 16
CHUNK = 80
DH = 128
NBUF = 4
NW = 8


def _mm1_body(x_ref, w_ref, o_ref):
    o_ref[...] = jnp.dot(x_ref[...], w_ref[...],
                         preferred_element_type=jnp.float32)


def _mm2_body(p0_ref, p1_ref, w_ref, o_ref):
    xv = jnp.maximum(p0_ref[...] + p1_ref[...], 0.0)
    o_ref[...] = jnp.dot(xv, w_ref[...], preferred_element_type=jnp.float32)


def _matmul1(x, w, block_rows=1000):
    n, k = x.shape
    m = w.shape[1]
    return pl.pallas_call(
        _mm1_body,
        grid=(n // block_rows,),
        in_specs=[
            pl.BlockSpec((block_rows, k), lambda i: (i, 0)),
            pl.BlockSpec((k, m), lambda i: (0, 0)),
        ],
        out_specs=pl.BlockSpec((block_rows, m), lambda i: (i, 0)),
        out_shape=jax.ShapeDtypeStruct((n, m), jnp.float32),
    )(x, w)


def _matmul2(p0, p1, w, block_rows=1000):
    n, k = p0.shape
    m = w.shape[1]
    return pl.pallas_call(
        _mm2_body,
        grid=(n // block_rows,),
        in_specs=[
            pl.BlockSpec((block_rows, k), lambda i: (i, 0)),
            pl.BlockSpec((block_rows, k), lambda i: (i, 0)),
            pl.BlockSpec((k, m), lambda i: (0, 0)),
        ],
        out_specs=pl.BlockSpec((block_rows, m), lambda i: (i, 0)),
        out_shape=jax.ShapeDtypeStruct((n, m), jnp.float32),
    )(p0, p1, w)


def _sc_aggregate(table, comb, binit, *, n_nodes, ch):
    mesh = plsc.VectorSubcoreMesh(core_axis_name="c", subcore_axis_name="s")
    npad = -(-n_nodes // (NS * 8)) * (NS * 8)
    rps = npad // NS
    assert ch % NW == 0 and ch >= NW

    @functools.partial(
        pl.kernel,
        mesh=mesh,
        out_type=jax.ShapeDtypeStruct((2, npad, DH), jnp.float32),
        scratch_types=[
            pltpu.VMEM((NW, 3, CHUNK), jnp.int32),
            pltpu.VMEM((NBUF, CHUNK, DH), jnp.float32),
            pltpu.VMEM((DH,), jnp.float32),
            pltpu.VMEM_SHARED((npad, DH), jnp.float32),
        ] + [pltpu.SemaphoreType.DMA] * (2 * NBUF + NW),
    )
    def agg_kernel(tab_h, comb_h, b_h, out_h,
                   comb_v, rows_v, b_v, acc, *sems):
        gsem = sems[:NBUF]
        ssem = sems[NBUF:2 * NBUF]
        csem = sems[2 * NBUF:]
        c = lax.axis_index("c")
        s = lax.axis_index("s")
        pltpu.sync_copy(b_h.at[c], b_v)

        def fill_row(r, _):
            for kk in range(DH // L):
                rows_v[0, r, pl.ds(kk * L, L)] = b_v[pl.ds(kk * L, L)]
            return 0
        lax.fori_loop(0, CHUNK, fill_row, 0)
        r0 = s * rps
        off = 0
        while off < rps:
            nr = min(CHUNK, rps - off)
            pltpu.sync_copy(rows_v.at[0, pl.ds(0, nr)],
                            acc.at[pl.ds(r0 + off, nr)])
            off += nr
        plsc.subcore_barrier()

        bcast_dnums = lax.GatherDimensionNumbers(
            offset_dims=(), collapsed_slice_dims=(0,), start_index_map=(0,))

        def comb_load(j, w):
            jc = jnp.minimum(j, ch - 1)
            pltpu.async_copy(comb_h.at[c, s, jc], comb_v.at[w], csem[w])

        def comb_wait(w):
            pltpu.make_async_copy(comb_h.at[0, 0, 0], comb_v.at[w],
                                  csem[w]).wait()

        def gather_start(b, w):
            pltpu.async_copy(tab_h.at[comb_v.at[w, 0]], rows_v.at[b],
                             gsem[b])

        def gather_wait(b):
            pltpu.make_async_copy(tab_h.at[comb_v.at[0, 0]], rows_v.at[b],
                                  gsem[b]).wait()

        def scatter_start(b, w):
            pltpu.async_copy(rows_v.at[b], acc.at[comb_v.at[w, 1]],
                             ssem[b], add=True)

        def scatter_wait(b):
            pltpu.make_async_copy(rows_v.at[b], acc.at[comb_v.at[0, 1]],
                                  ssem[b]).wait()

        def scale_rows(b, w):
            def group_body(g, _):
                eav16 = lax.bitcast_convert_type(
                    comb_v[w, 2, pl.ds(g * L, L)], jnp.float32)
                for t in range(L):
                    eab = lax.gather(
                        eav16, jnp.full((L, 1), t, jnp.int32), bcast_dnums,
                        (1,), mode=lax.GatherScatterMode.PROMISE_IN_BOUNDS)
                    i = g * L + t
                    rv = rows_v.at[b, i]
                    for kk in range(DH // L):
                        rv[pl.ds(kk * L, L)] = rv[pl.ds(kk * L, L)] * eab
                return 0
            lax.fori_loop(0, CHUNK // L, group_body, 0)

        def position(q, qq):
            b, w = qq % NBUF, qq % NW
            if not (isinstance(q, int) and q < 3):
                scatter_wait((qq + 1) % NBUF)
            comb_load(q + 5, (qq + 5) % NW)
            comb_wait((qq + 1) % NW)
            gather_start((qq + 1) % NBUF, (qq + 1) % NW)
            gather_wait(b)
            scale_rows(b, w)
            scatter_start(b, w)

        for j in range(5):
            comb_load(j, j)
        comb_wait(0)
        gather_start(0, 0)
        for qq in range(NW):
            position(qq, qq)

        def cycle_body(k, _):
            for qq in range(NW):
                position(k * NW + qq, qq)
            return 0
        lax.fori_loop(1, ch // NW, cycle_body, 0)

        for w in range(1, 5):
            comb_wait(w)
        gather_wait(0)
        for b in range(1, NBUF):
            scatter_wait(b)
        plsc.subcore_barrier()
        pltpu.sync_copy(acc.at[pl.ds(r0, rps)], out_h.at[c, pl.ds(r0, rps)])

    return agg_kernel(table, comb, binit)


def _pad_reshape(a, ep, shape):
    return jnp.pad(a, (0, ep - a.shape[0])).reshape(shape)


def kernel(x, edge_index, edge_attr, W1, b1, W2, b2):
    n = x.shape[0]
    d_hid = W1.shape[1]
    d_out = W2.shape[1]
    src = edge_index[0].astype(jnp.int32)
    dst = edge_index[1].astype(jnp.int32)
    ea = edge_attr.astype(jnp.float32)
    e = src.shape[0]

    eab = jax.lax.bitcast_convert_type(ea, jnp.int32)

    ep1 = -(-e // (NC * NS * CHUNK * NW)) * (NC * NS * CHUNK * NW)
    ch1 = ep1 // (NC * NS * CHUNK)
    sh1 = (NC, NS, ch1, CHUNK)
    comb1 = jnp.stack([_pad_reshape(src, ep1, sh1),
                       _pad_reshape(dst, ep1, sh1),
                       _pad_reshape(eab, ep1, sh1)], axis=3)
    b1init = jnp.stack([b1, jnp.zeros_like(b1)])

    ep2 = -(-e // (NS * CHUNK * NW)) * (NS * CHUNK * NW)
    ch2 = ep2 // (NS * CHUNK)
    sh2 = (NS, ch2, CHUNK)
    src2 = _pad_reshape(src, ep2, sh2)
    idx2 = 2 * src2[None] + jnp.arange(NC, dtype=jnp.int32)[:, None, None, None]
    dst2 = jnp.broadcast_to(_pad_reshape(dst, ep2, sh2)[None], (NC,) + sh2)
    ea2 = jnp.broadcast_to(_pad_reshape(eab, ep2, sh2)[None], (NC,) + sh2)
    comb2 = jnp.stack([idx2, dst2, ea2], axis=3)
    b2init = b2.reshape(NC, d_out // NC)

    h1 = _matmul1(x, W1)
    parts = _sc_aggregate(h1, comb1, b1init, n_nodes=n, ch=ch1)
    h2 = _matmul2(parts[0, :n], parts[1, :n], W2)
    halves = _sc_aggregate(h2.reshape(2 * n, d_out // 2), comb2,
                           b2init, n_nodes=n, ch=ch2)
    return halves[:, :n, :].transpose(1, 0, 2).reshape(n, d_out)
